# Initial kernel scaffold; baseline (speedup 1.0000x reference)
import jax
import jax.numpy as jnp
from jax.experimental import pallas as pl


def kernel(x_users, x_artists, edge_index_a2u, edge_index_u2a):
    raise NotImplementedError("write your pallas kernel here")


# speedup vs baseline: 2.5964x; 2.5964x over previous
import functools

import jax
import jax.numpy as jnp
from jax import lax
from jax.experimental import pallas as pl
from jax.experimental.pallas import tpu as pltpu
from jax.experimental.pallas import tpu_sc as plsc

NU = 50000
NE = 600000
DC = 32
NCHUNK = 4

NT = 51200
SENT = 50000
EP = 600064
EB = 128
EPT = EP // 16
NBLK = EPT // EB
RPT = NT // 16
RB = 160
NRB = RPT // RB
CW = 16

_MESH = plsc.VectorSubcoreMesh(
    core_axis_name="c", subcore_axis_name="s", num_cores=2, num_subcores=16
)

_f32 = jnp.float32
_i32 = jnp.int32


def _zero_fill(ref, nrows, width):
    def body(r, _):
        for p in range(width // 16):
            ref[r, pl.ds(16 * p, 16)] = jnp.zeros((16,), _f32)
        return _
    lax.fori_loop(0, nrows, body, None)


def _counts_body(da_hbm, du_hbm, invu_hbm, inva_hbm,
                 cnt_sh, didx_v, ones_v, czero_v, cstage_v, invstage_v):
    cid = lax.axis_index("c")
    sid = lax.axis_index("s")

    def onesfill(r, _):
        ones_v[r, pl.ds(0, CW)] = jnp.ones((CW,), _f32)
        return _
    lax.fori_loop(0, EB, onesfill, None)
    _zero_fill(czero_v, RB, CW)

    for sel, dst_hbm, inv_out in ((0, da_hbm, invu_hbm), (1, du_hbm, inva_hbm)):
        @pl.when(cid == sel)
        def _(dst_hbm=dst_hbm, inv_out=inv_out):
            def zacc(rb, _):
                pltpu.sync_copy(
                    czero_v, cnt_sh.at[pl.ds(sid * RPT + rb * RB, RB), :])
                return _
            lax.fori_loop(0, NRB, zacc, None)
            plsc.subcore_barrier()

            def ebody(b, _):
                off = sid * EPT + b * EB
                pltpu.sync_copy(dst_hbm.at[pl.ds(off, EB)], didx_v)
                pltpu.sync_copy(ones_v, cnt_sh.at[didx_v], add=True)
                return _
            lax.fori_loop(0, NBLK, ebody, None)
            plsc.subcore_barrier()

            def wbody(rb, _):
                r0 = sid * RPT + rb * RB
                pltpu.sync_copy(cnt_sh.at[pl.ds(r0, RB), :], cstage_v)
                def rbody(r, _):
                    c = cstage_v[r, pl.ds(0, CW)]
                    invstage_v[r, pl.ds(0, CW)] = 1.0 / jnp.maximum(c, 1.0)
                    return _
                lax.fori_loop(0, RB, rbody, None)
                pltpu.sync_copy(invstage_v, inv_out.at[pl.ds(r0, RB), :])
                return _
            lax.fori_loop(0, NRB, wbody, None)
            plsc.subcore_barrier()


_SC_PARAMS = pltpu.CompilerParams(use_tc_tiling_on_sc=False)

_counts = pl.kernel(
    _counts_body,
    out_type=[jax.ShapeDtypeStruct((NT, CW), _f32)] * 2,
    mesh=_MESH,
    compiler_params=_SC_PARAMS,
    scratch_types=[
        pltpu.VMEM_SHARED((NT, CW), _f32),
        pltpu.VMEM((EB,), _i32),
        pltpu.VMEM((EB, CW), _f32),
        pltpu.VMEM((RB, CW), _f32),
        pltpu.VMEM((RB, CW), _f32),
        pltpu.VMEM((RB, CW), _f32),
    ],
)


def _pass_body(src_hbm, dst_hbm, t0, t1, t2, t3, inv_hbm, f0, f1, f2, f3,
               scale_hbm,
               o0, o1, o2, o3, g0, g1, g2, g3,
               acc_sh, sidx_v, didx_v, rows_v, stage_v, fin_v, inv_v,
               scale_v, zero_v):
    cid = lax.axis_index("c")
    sid = lax.axis_index("s")
    pltpu.sync_copy(scale_hbm, scale_v)
    scale_reg = scale_v[...]
    _zero_fill(zero_v, RB, DC)

    tabs = (t0, t1, t2, t3)
    fins = (f0, f1, f2, f3)
    outs = (o0, o1, o2, o3)
    gouts = (g0, g1, g2, g3)

    for ch in range(NCHUNK):
        @pl.when(cid == (ch % 2))
        def _(tbl=tabs[ch], fsrc=fins[ch], out=outs[ch], gout=gouts[ch]):
            def zacc(rb, _):
                pltpu.sync_copy(
                    zero_v, acc_sh.at[pl.ds(sid * RPT + rb * RB, RB), :])
                return _
            lax.fori_loop(0, NRB, zacc, None)
            plsc.subcore_barrier()

            def ebody(b, _):
                off = sid * EPT + b * EB
                pltpu.sync_copy(src_hbm.at[pl.ds(off, EB)], sidx_v)
                pltpu.sync_copy(dst_hbm.at[pl.ds(off, EB)], didx_v)
                pltpu.sync_copy(tbl.at[sidx_v], rows_v)
                pltpu.sync_copy(rows_v, acc_sh.at[didx_v], add=True)
                return _
            lax.fori_loop(0, NBLK, ebody, None)
            plsc.subcore_barrier()

            def wbody(rb, _):
                r0 = sid * RPT + rb * RB
                pltpu.sync_copy(acc_sh.at[pl.ds(r0, RB), :], stage_v)
                pltpu.sync_copy(fsrc.at[pl.ds(r0, RB), :], fin_v)
                pltpu.sync_copy(inv_hbm.at[pl.ds(r0, RB), :], inv_v)
                def rbody(r, _):
                    iv = inv_v[r, pl.ds(0, CW)]
                    for p in range(DC // 16):
                        sl = pl.ds(16 * p, 16)
                        m = stage_v[r, sl] * iv
                        stage_v[r, sl] = m
                        fin_v[r, sl] = fin_v[r, sl] * scale_reg + m * 0.25
                    return _
                lax.fori_loop(0, RB, rbody, None)
                pltpu.sync_copy(stage_v, out.at[pl.ds(r0, RB), :])
                pltpu.sync_copy(fin_v, gout.at[pl.ds(r0, RB), :])
                return _
            lax.fori_loop(0, NRB, wbody, None)
            plsc.subcore_barrier()


_pass = pl.kernel(
    _pass_body,
    out_type=[jax.ShapeDtypeStruct((NT, DC), _f32)] * 8,
    mesh=_MESH,
    compiler_params=_SC_PARAMS,
    scratch_types=[
        pltpu.VMEM_SHARED((NT, DC), _f32),
        pltpu.VMEM((EB,), _i32),
        pltpu.VMEM((EB,), _i32),
        pltpu.VMEM((EB, DC), _f32),
        pltpu.VMEM((RB, DC), _f32),
        pltpu.VMEM((RB, DC), _f32),
        pltpu.VMEM((RB, CW), _f32),
        pltpu.VMEM((16,), _f32),
        pltpu.VMEM((RB, DC), _f32),
    ],
)


def kernel(x_users, x_artists, edge_index_a2u, edge_index_u2a):
    pad = EP - NE

    def prep_edges(e):
        src = jnp.concatenate(
            [e[0].astype(_i32), jnp.zeros((pad,), _i32)])
        dst = jnp.concatenate(
            [e[1].astype(_i32), jnp.full((pad,), SENT, _i32)])
        return src, dst

    sa, da = prep_edges(edge_index_a2u)
    su, du = prep_edges(edge_index_u2a)

    def chunks(x):
        xp = jnp.pad(x.astype(_f32), ((0, NT - NU), (0, 0)))
        return [xp[:, DC * c:DC * (c + 1)] for c in range(NCHUNK)]

    xu_c = chunks(x_users)
    xa_c = chunks(x_artists)

    inv_u, inv_a = _counts(da, du)

    scale_first = jnp.full((16,), 0.25, _f32)
    scale_rest = jnp.ones((16,), _f32)

    cu, ca = xu_c, xa_c
    fu, fa = xu_c, xa_c
    for layer in range(3):
        sc = scale_first if layer == 0 else scale_rest
        res = _pass(sa, da, *ca, inv_u, *fu, sc)
        cu, fu = res[:4], res[4:]
        res = _pass(su, du, *cu, inv_a, *fa, sc)
        ca, fa = res[:4], res[4:]

    final_u = jnp.concatenate([f[:NU] for f in fu], axis=1)
    final_a = jnp.concatenate([f[:NU] for f in fa], axis=1)
    return (final_u, final_a)
